# baseline (speedup 1.0000x reference)
import functools

import jax
import jax.numpy as jnp
from jax import lax
from jax.experimental import pallas as pl
from jax.experimental.pallas import tpu as pltpu
from jax.experimental.pallas import tpu_sc as plsc

_B = 4096
_S = 50
_DIM = 65
_ODIM = 80
_ROWS = _B * _S
_NODES = 1000000
_PDIM = 128
_BN = 16384
_GRID = (_NODES + _BN - 1) // _BN
_NPAD = _GRID * _BN
_NC = 2
_NS = 16
_NW = _NC * _NS
_RPW = _ROWS // _NW
_CH = 128
_NCH = _RPW // _CH
_NPAIR = _NCH // 2


def _transpose_body(in_ref, out_ref):
    x = in_ref[...]
    eye = jnp.eye(_DIM, _PDIM, dtype=jnp.float32)
    out_ref[...] = lax.dot_general(
        x, eye, (((0,), (0,)), ((), ())), preferred_element_type=jnp.float32
    )


_transpose = pl.pallas_call(
    _transpose_body,
    grid=(_GRID,),
    in_specs=[pl.BlockSpec((_DIM, _BN), lambda i: (0, i))],
    out_specs=pl.BlockSpec((_BN, _PDIM), lambda i: (i, 0)),
    out_shape=jax.ShapeDtypeStruct((_NPAD, _PDIM), jnp.float32),
)


def _make_gather():
    mesh = plsc.VectorSubcoreMesh(core_axis_name="c", subcore_axis_name="s")

    @functools.partial(
        pl.kernel,
        mesh=mesh,
        compiler_params=pltpu.CompilerParams(use_tc_tiling_on_sc=True),
        out_type=jax.ShapeDtypeStruct((_ROWS, _PDIM), jnp.float32),
        scratch_types=[
            pltpu.VMEM((_CH,), jnp.int32),
            pltpu.VMEM((_CH,), jnp.int32),
            pltpu.VMEM((_CH, _PDIM), jnp.float32),
            pltpu.VMEM((_CH, _PDIM), jnp.float32),
            pltpu.SemaphoreType.DMA,
            pltpu.SemaphoreType.DMA,
            pltpu.SemaphoreType.DMA,
            pltpu.SemaphoreType.DMA,
        ],
    )
    def gather_kernel(table_hbm, idx_hbm, out_hbm, idx0, idx1, rows0, rows1,
                      sg0, sg1, so0, so1):
        wid = lax.axis_index("s") * _NC + lax.axis_index("c")
        base = wid * _RPW

        def g_start(idx_v, rows_v, sem):
            pltpu.async_copy(table_hbm.at[idx_v], rows_v, sem)

        def g_wait(idx_v, rows_v, sem):
            pltpu.make_async_copy(table_hbm.at[idx_v], rows_v, sem).wait()

        def o_start(rows_v, pos, sem):
            pltpu.async_copy(rows_v, out_hbm.at[pl.ds(pos, _CH)], sem)

        def o_wait(rows_v, pos, sem):
            pltpu.make_async_copy(
                rows_v, out_hbm.at[pl.ds(pos, _CH)], sem
            ).wait()

        pltpu.sync_copy(idx_hbm.at[pl.ds(base, _CH)], idx0)
        g_start(idx0, rows0, sg0)

        def body(i, carry):
            t0 = 2 * i
            p0 = base + t0 * _CH
            p1 = p0 + _CH
            pltpu.sync_copy(idx_hbm.at[pl.ds(p1, _CH)], idx1)
            g_wait(idx0, rows0, sg0)

            @pl.when(i > 0)
            def _():
                o_wait(rows1, p0 - _CH, so1)

            g_start(idx1, rows1, sg1)
            o_start(rows0, p0, so0)

            @pl.when(i < _NPAIR - 1)
            def _():
                pltpu.sync_copy(idx_hbm.at[pl.ds(p1 + _CH, _CH)], idx0)

            g_wait(idx1, rows1, sg1)
            o_wait(rows0, p0, so0)

            @pl.when(i < _NPAIR - 1)
            def _():
                g_start(idx0, rows0, sg0)

            o_start(rows1, p1, so1)
            return carry

        lax.fori_loop(0, _NPAIR, body, 0)
        o_wait(rows1, base + (_NCH - 1) * _CH, so1)

    return gather_kernel


_gather = _make_gather()


def kernel(idx, embedding):
    idx_flat = idx.reshape(_ROWS).astype(jnp.int32)
    table128 = _transpose(embedding.T)
    out128 = _gather(table128, idx_flat)
    return out128[:, :_DIM].reshape(_B, _S, _DIM)


# speedup vs baseline: 1.0128x; 1.0128x over previous
import functools

import jax
import jax.numpy as jnp
from jax import lax
from jax.experimental import pallas as pl
from jax.experimental.pallas import tpu as pltpu
from jax.experimental.pallas import tpu_sc as plsc

_B = 4096
_S = 50
_DIM = 65
_ODIM = 80
_ROWS = _B * _S
_NODES = 1000000
_PDIM = 128
_BN = 32768
_GRID = (_NODES + _BN - 1) // _BN
_NPAD = _GRID * _BN
_NC = 2
_NS = 16
_NW = _NC * _NS
_RPW = _ROWS // _NW
_CH = 128
_NCH = _RPW // _CH
_NPAIR = _NCH // 2


def _transpose_body(in_ref, out_ref):
    x = in_ref[...]
    y = jnp.transpose(x, (1, 0))
    out_ref[:, :_DIM] = y
    out_ref[:, _DIM:] = jnp.zeros((_BN, _PDIM - _DIM), jnp.float32)


_transpose = pl.pallas_call(
    _transpose_body,
    grid=(_GRID,),
    in_specs=[pl.BlockSpec((_DIM, _BN), lambda i: (0, i))],
    out_specs=pl.BlockSpec((_BN, _PDIM), lambda i: (i, 0)),
    out_shape=jax.ShapeDtypeStruct((_NPAD, _PDIM), jnp.float32),
)


def _make_gather():
    mesh = plsc.VectorSubcoreMesh(core_axis_name="c", subcore_axis_name="s")

    @functools.partial(
        pl.kernel,
        mesh=mesh,
        compiler_params=pltpu.CompilerParams(use_tc_tiling_on_sc=True),
        out_type=jax.ShapeDtypeStruct((_ROWS, _PDIM), jnp.float32),
        scratch_types=[
            pltpu.VMEM((_CH,), jnp.int32),
            pltpu.VMEM((_CH,), jnp.int32),
            pltpu.VMEM((_CH, _PDIM), jnp.float32),
            pltpu.VMEM((_CH, _PDIM), jnp.float32),
            pltpu.SemaphoreType.DMA,
            pltpu.SemaphoreType.DMA,
            pltpu.SemaphoreType.DMA,
            pltpu.SemaphoreType.DMA,
        ],
    )
    def gather_kernel(table_hbm, idx_hbm, out_hbm, idx0, idx1, rows0, rows1,
                      sg0, sg1, so0, so1):
        wid = lax.axis_index("s") * _NC + lax.axis_index("c")
        base = wid * _RPW

        def g_start(idx_v, rows_v, sem):
            pltpu.async_copy(table_hbm.at[idx_v], rows_v, sem)

        def g_wait(idx_v, rows_v, sem):
            pltpu.make_async_copy(table_hbm.at[idx_v], rows_v, sem).wait()

        def o_start(rows_v, pos, sem):
            pltpu.async_copy(rows_v, out_hbm.at[pl.ds(pos, _CH)], sem)

        def o_wait(rows_v, pos, sem):
            pltpu.make_async_copy(
                rows_v, out_hbm.at[pl.ds(pos, _CH)], sem
            ).wait()

        pltpu.sync_copy(idx_hbm.at[pl.ds(base, _CH)], idx0)
        g_start(idx0, rows0, sg0)

        def body(i, carry):
            t0 = 2 * i
            p0 = base + t0 * _CH
            p1 = p0 + _CH
            pltpu.sync_copy(idx_hbm.at[pl.ds(p1, _CH)], idx1)
            g_wait(idx0, rows0, sg0)

            @pl.when(i > 0)
            def _():
                o_wait(rows1, p0 - _CH, so1)

            g_start(idx1, rows1, sg1)
            o_start(rows0, p0, so0)

            @pl.when(i < _NPAIR - 1)
            def _():
                pltpu.sync_copy(idx_hbm.at[pl.ds(p1 + _CH, _CH)], idx0)

            g_wait(idx1, rows1, sg1)
            o_wait(rows0, p0, so0)

            @pl.when(i < _NPAIR - 1)
            def _():
                g_start(idx0, rows0, sg0)

            o_start(rows1, p1, so1)
            return carry

        lax.fori_loop(0, _NPAIR, body, 0)
        o_wait(rows1, base + (_NCH - 1) * _CH, so1)

    return gather_kernel


_gather = _make_gather()


def kernel(idx, embedding):
    idx_flat = idx.reshape(_ROWS).astype(jnp.int32)
    table128 = _transpose(embedding.T)
    out128 = _gather(table128, idx_flat)
    return out128[:, :_DIM].reshape(_B, _S, _DIM)
